# baseline (speedup 1.0000x reference)
import functools

import jax
import jax.numpy as jnp
from jax import lax
from jax.experimental import pallas as pl
from jax.experimental.pallas import tpu as pltpu
from jax.experimental.pallas import tpu_sc as plsc

VOCAB = 1_000_000
EMB = 64
CTX = 2
B = 1024
L = 20

NC = 2
NS = 16
NW = NC * NS
ROWS_PER_W = B // NW
IDX_PER_W = ROWS_PER_W * L
LANES = 16
KCOL = EMB // LANES
CB = 2
CTOK = CB * L
NCH = ROWS_PER_W // CB
TGRP = 8


def _cbow_body(x_hbm, w_hbm, out_hbm, idx_v, rows_v, w0_v, out_v,
               sem0, sem1):
    wid = lax.axis_index("s") * NC + lax.axis_index("c")

    pltpu.sync_copy(x_hbm.at[wid], idx_v.at[pl.ds(0, IDX_PER_W)])
    pltpu.sync_copy(w_hbm.at[pl.ds(0, 1)], w0_v)

    sems = [sem0, sem1]

    def fire(c, p):

        def grp(g, cc):
            tv = idx_v[pl.ds(c * CTOK + g * TGRP, LANES)]
            for l in range(TGRP):
                tid = lax.shift_right_logical(tv[l], 3)
                pltpu.async_copy(
                    w_hbm.at[pl.ds(tid, 1)],
                    rows_v.at[p, pl.ds(g * TGRP + l, 1)],
                    sems[p],
                )
            return cc

        lax.fori_loop(0, CTOK // TGRP, grp, 0)

    def drain(p):
        pltpu.make_async_copy(
            w_hbm.at[pl.ds(0, CTOK)], rows_v.at[p], sems[p]
        ).wait()

    w0s = [
        jnp.reshape(
            w0_v[pl.ds(0, 1), pl.ds(0, 1), pl.ds(k * LANES, LANES)], (LANES,)
        )
        for k in range(KCOL)
    ]

    def compute(c, p):
        def body(i, cc):
            b = c * CB + i
            lbase = i * L
            tv0 = idx_v[pl.ds(c * CTOK + lbase, LANES)]
            tv1 = idx_v[pl.ds(c * CTOK + lbase + 4, LANES)]
            offs = []
            for j in range(L):
                tok = tv0[j] if j < LANES else tv1[j - 4]
                offs.append(tok & (TGRP - 1))

            def rload(j, k):
                return jnp.reshape(
                    rows_v[p, pl.ds(lbase + j, 1), pl.ds(offs[j], 1),
                           pl.ds(k * LANES, LANES)],
                    (LANES,),
                )

            for k in range(KCOL):
                col = pl.ds(k * LANES, LANES)
                r = [rload(j, k) for j in range(L)]
                w0 = w0s[k]
                s = r[0]
                for j in range(1, L):
                    s = s + r[j]
                t = s + w0
                o0 = t - r[L - 1]
                o1 = o0 + w0 - r[L - 2]
                o2 = t - r[0]
                o3 = o2 + w0 - r[1]
                out_v[b, 0, col] = o0
                out_v[b, 1, col] = o1
                out_v[b, 2, col] = o2
                out_v[b, 3, col] = o3
            return cc

        lax.fori_loop(0, CB, body, 0)

    fire(0, 0)
    fire(1, 1)

    def pair(h, carry):
        c0 = 2 * h
        drain(0)
        compute(c0, 0)
        fire(c0 + 2, 0)
        drain(1)
        compute(c0 + 1, 1)
        fire(c0 + 3, 1)
        return carry

    lax.fori_loop(0, NCH // 2 - 1, pair, 0)
    drain(0)
    compute(NCH - 2, 0)
    drain(1)
    compute(NCH - 1, 1)

    pltpu.sync_copy(out_v, out_hbm.at[pl.ds(wid * ROWS_PER_W, ROWS_PER_W)])


def kernel(x, W):
    x2 = x.reshape(NW, IDX_PER_W).astype(jnp.int32)
    W3 = W.reshape(VOCAB // TGRP, TGRP, EMB)
    mesh = plsc.VectorSubcoreMesh(core_axis_name="c", subcore_axis_name="s")
    f = functools.partial(
        pl.kernel,
        mesh=mesh,
        out_type=jax.ShapeDtypeStruct((B, 2 * CTX, EMB), jnp.float32),
        scratch_types=[
            pltpu.VMEM((IDX_PER_W + LANES,), jnp.int32),
            pltpu.VMEM((2, CTOK, TGRP, EMB), jnp.float32),
            pltpu.VMEM((1, TGRP, EMB), jnp.float32),
            pltpu.VMEM((ROWS_PER_W, 2 * CTX, EMB), jnp.float32),
            pltpu.SemaphoreType.DMA,
            pltpu.SemaphoreType.DMA,
        ],
        compiler_params=pltpu.CompilerParams(use_tc_tiling_on_sc=True),
    )(_cbow_body)
    return f(x2, W3)


# speedup vs baseline: 1.1080x; 1.1080x over previous
import functools

import jax
import jax.numpy as jnp
from jax import lax
from jax.experimental import pallas as pl
from jax.experimental.pallas import tpu as pltpu
from jax.experimental.pallas import tpu_sc as plsc

VOCAB = 1_000_000
EMB = 64
CTX = 2
B = 1024
L = 20

NC = 2
NS = 16
NW = NC * NS
ROWS_PER_W = B // NW
IDX_PER_W = ROWS_PER_W * L
LANES = 16
KCOL = EMB // LANES
TGRP = 8
FGRP = 16


def _cbow_body(x_hbm, w_hbm, out_hbm, idx_v, rows_v, w0_v, out_v, sem):
    wid = lax.axis_index("s") * NC + lax.axis_index("c")

    pltpu.sync_copy(x_hbm.at[wid], idx_v.at[pl.ds(0, IDX_PER_W)])
    pltpu.sync_copy(w_hbm.at[0, pl.ds(0, 1)], w0_v)

    def fire(g, cc):
        tv = idx_v[pl.ds(g * FGRP, FGRP)]
        for l in range(FGRP):
            tok = tv[l]
            tid = lax.shift_right_logical(tok, 3)
            w = tok & (TGRP - 1)
            pltpu.async_copy(
                w_hbm.at[tid, pl.ds(w, 1)],
                rows_v.at[pl.ds(g * FGRP + l, 1)],
                sem,
            )
        return cc

    lax.fori_loop(0, IDX_PER_W // FGRP, fire, 0)

    def drain(t, cc):
        pltpu.make_async_copy(
            w_hbm.at[0, pl.ds(0, 1)], rows_v.at[pl.ds(t, 1)], sem
        ).wait()
        return cc

    lax.fori_loop(0, IDX_PER_W, drain, 0)

    w0s = [
        jnp.reshape(w0_v[pl.ds(0, 1), pl.ds(k * LANES, LANES)], (LANES,))
        for k in range(KCOL)
    ]

    def body(b, cc):
        base = b * L

        def rload(j, k):
            return jnp.reshape(
                rows_v[pl.ds(base + j, 1), pl.ds(k * LANES, LANES)],
                (LANES,),
            )

        for k in range(KCOL):
            col = pl.ds(k * LANES, LANES)
            r = [rload(j, k) for j in range(L)]
            w0 = w0s[k]
            s = r[0]
            for j in range(1, L):
                s = s + r[j]
            t = s + w0
            o0 = t - r[L - 1]
            o1 = o0 + w0 - r[L - 2]
            o2 = t - r[0]
            o3 = o2 + w0 - r[1]
            out_v[b, 0, col] = o0
            out_v[b, 1, col] = o1
            out_v[b, 2, col] = o2
            out_v[b, 3, col] = o3
        return cc

    lax.fori_loop(0, ROWS_PER_W, body, 0)

    pltpu.sync_copy(out_v, out_hbm.at[pl.ds(wid * ROWS_PER_W, ROWS_PER_W)])


def kernel(x, W):
    x2 = x.reshape(NW, IDX_PER_W).astype(jnp.int32)
    W3 = W.reshape(VOCAB // TGRP, TGRP, EMB)
    mesh = plsc.VectorSubcoreMesh(core_axis_name="c", subcore_axis_name="s")
    f = functools.partial(
        pl.kernel,
        mesh=mesh,
        out_type=jax.ShapeDtypeStruct((B, 2 * CTX, EMB), jnp.float32),
        scratch_types=[
            pltpu.VMEM((IDX_PER_W + LANES,), jnp.int32),
            pltpu.VMEM((IDX_PER_W, EMB), jnp.float32),
            pltpu.VMEM((1, EMB), jnp.float32),
            pltpu.VMEM((ROWS_PER_W, 2 * CTX, EMB), jnp.float32),
            pltpu.SemaphoreType.DMA,
        ],
        compiler_params=pltpu.CompilerParams(use_tc_tiling_on_sc=True),
    )(_cbow_body)
    return f(x2, W3)
